# baseline (speedup 1.0000x reference)
import functools

import jax
import jax.numpy as jnp
from jax import lax
from jax.experimental import pallas as pl
from jax.experimental.pallas import tpu as pltpu
from jax.experimental.pallas import tpu_sc as plsc

B = 512
D = 128
K = 1024

NC = 2
NS = 16
L = 16
NW = NC * NS
BPW = B // NW


def _dist_body(x_ref, w_ref, d2t_out, loss_out):
    x = x_ref[:]
    v = w_ref[:] - 1e-6
    dots_t = lax.dot_general(
        v, x, (((0,), (1,)), ((), ())),
        preferred_element_type=jnp.float32,
        precision=lax.Precision.HIGHEST,
    )
    vsq = jnp.sum(v * v, axis=0)[:, None]
    xsq = jnp.sum(x * x, axis=1)[None, :]
    d2t = jnp.maximum(vsq + xsq - 2.0 * dots_t, 0.0)
    d2t_out[:] = d2t
    mind2 = jnp.min(d2t, axis=0, keepdims=True)
    loss_out[0, 0] = jnp.sum(jnp.sqrt(mind2)) / B


@functools.partial(
    pl.kernel,
    out_type=(
        jax.ShapeDtypeStruct((B, 2), jnp.float32),
        jax.ShapeDtypeStruct((B,), jnp.int32),
    ),
    mesh=plsc.VectorSubcoreMesh(core_axis_name="c", subcore_axis_name="s"),
    compiler_params=pltpu.CompilerParams(use_tc_tiling_on_sc=False,
                                          needs_layout_passes=False),
    scratch_types=[
        pltpu.VMEM((K, BPW), jnp.float32),
        pltpu.VMEM((K, 2), jnp.float32),
        pltpu.VMEM((BPW, 2), jnp.float32),
        pltpu.VMEM((BPW,), jnp.int32),
    ],
)
def _sc_bmu(d2t_hbm, loc_hbm, locs_hbm, idx_hbm, slab_v, loc_v, locs_v, idx_v):
    wid = lax.axis_index("s") * NC + lax.axis_index("c")
    base = wid * BPW
    pltpu.sync_copy(d2t_hbm.at[:, pl.ds(base, BPW)], slab_v)
    pltpu.sync_copy(loc_hbm, loc_v)

    def step(k, carry):
        mv, mi = carry
        row = slab_v[k, :]
        pred = row < mv
        mv = jnp.where(pred, row, mv)
        mi = jnp.where(pred, jnp.full((L,), k, jnp.int32), mi)
        return mv, mi

    mv0 = jnp.full((L,), jnp.inf, jnp.float32)
    mi0 = jnp.zeros((L,), jnp.int32)
    _, mi = lax.fori_loop(0, K, step, (mv0, mi0))

    iota = lax.iota(jnp.int32, L)
    zeros = jnp.zeros((L,), jnp.int32)
    ones = jnp.ones((L,), jnp.int32)
    lx = plsc.load_gather(loc_v, [mi, zeros])
    ly = plsc.load_gather(loc_v, [mi, ones])
    plsc.store_scatter(locs_v, [iota, zeros], lx)
    plsc.store_scatter(locs_v, [iota, ones], ly)
    idx_v[...] = mi
    pltpu.sync_copy(locs_v, locs_hbm.at[pl.ds(base, BPW), :])
    pltpu.sync_copy(idx_v, idx_hbm.at[pl.ds(base, BPW)])


def kernel(input, weight, locations):
    d2t, loss = pl.pallas_call(
        _dist_body,
        out_shape=(
            jax.ShapeDtypeStruct((K, B), jnp.float32),
            jax.ShapeDtypeStruct((1, 1), jnp.float32),
        ),
        out_specs=(
            pl.BlockSpec(memory_space=pltpu.VMEM),
            pl.BlockSpec(memory_space=pltpu.SMEM),
        ),
    )(input, weight)
    locs, idx = _sc_bmu(d2t, locations)
    return locs.reshape(B, 1, 2), loss.reshape(()), idx.reshape(B, 1)


# speedup vs baseline: 1.2116x; 1.2116x over previous
import functools

import jax
import jax.numpy as jnp
from jax import lax
from jax.experimental import pallas as pl
from jax.experimental.pallas import tpu as pltpu
from jax.experimental.pallas import tpu_sc as plsc

B = 512
D = 128
K = 1024

G = 16
KG = K // G

NC = 2
NS = 16
L = 16
NW = NC * NS
BPW = B // NW


def _dist_body(x_ref, w_ref, m64_out, j64_out, loss_out):
    x = x_ref[:]
    v = w_ref[:] - 1e-6
    dots_t = lax.dot_general(
        v, x, (((0,), (1,)), ((), ())),
        preferred_element_type=jnp.float32,
        precision=lax.Precision.HIGHEST,
    )
    vsq = jnp.sum(v * v, axis=0)[:, None]
    xsq = jnp.sum(x * x, axis=1)[None, :]
    d2t = jnp.maximum(vsq + xsq - 2.0 * dots_t, 0.0)
    d2g = d2t.reshape(G, KG, B)
    m64 = jnp.min(d2g, axis=1)
    joff = jax.lax.broadcasted_iota(jnp.int32, (G, KG, B), 1)
    j64 = jnp.min(jnp.where(d2g == m64[:, None, :], joff, KG), axis=1)
    m64_out[:] = m64
    j64_out[:] = j64
    mind2 = jnp.min(m64, axis=0, keepdims=True)
    loss_out[0, 0] = jnp.sum(jnp.sqrt(mind2)) / B


@functools.partial(
    pl.kernel,
    out_type=(
        jax.ShapeDtypeStruct((B, 2), jnp.float32),
        jax.ShapeDtypeStruct((B,), jnp.int32),
    ),
    mesh=plsc.VectorSubcoreMesh(core_axis_name="c", subcore_axis_name="s"),
    compiler_params=pltpu.CompilerParams(use_tc_tiling_on_sc=False,
                                         needs_layout_passes=False),
    scratch_types=[
        pltpu.VMEM((G, BPW), jnp.float32),
        pltpu.VMEM((G, BPW), jnp.int32),
        pltpu.VMEM((K, 2), jnp.float32),
        pltpu.VMEM((BPW, 2), jnp.float32),
        pltpu.VMEM((BPW,), jnp.int32),
    ],
)
def _sc_bmu(m64_hbm, j64_hbm, loc_hbm, locs_hbm, idx_hbm,
            m64_v, j64_v, loc_v, locs_v, idx_v):
    wid = lax.axis_index("s") * NC + lax.axis_index("c")
    base = wid * BPW
    pltpu.sync_copy(m64_hbm.at[:, pl.ds(base, BPW)], m64_v)
    pltpu.sync_copy(j64_hbm.at[:, pl.ds(base, BPW)], j64_v)
    pltpu.sync_copy(loc_hbm, loc_v)

    rows = [m64_v[g, :] for g in range(G)]
    mv = rows[0]
    for g in range(1, G):
        mv = jnp.minimum(mv, rows[g])
    gwin = jnp.full((L,), G - 1, jnp.int32)
    for g in range(G - 2, -1, -1):
        gwin = jnp.where(rows[g] == mv, jnp.full((L,), g, jnp.int32), gwin)

    iota = lax.iota(jnp.int32, L)
    joff = plsc.load_gather(j64_v, [gwin, iota])
    mi = gwin * KG + joff

    zeros = jnp.zeros((L,), jnp.int32)
    ones = jnp.ones((L,), jnp.int32)
    lx = plsc.load_gather(loc_v, [mi, zeros])
    ly = plsc.load_gather(loc_v, [mi, ones])
    plsc.store_scatter(locs_v, [iota, zeros], lx)
    plsc.store_scatter(locs_v, [iota, ones], ly)
    idx_v[...] = mi
    pltpu.sync_copy(locs_v, locs_hbm.at[pl.ds(base, BPW), :])
    pltpu.sync_copy(idx_v, idx_hbm.at[pl.ds(base, BPW)])


def kernel(input, weight, locations):
    m64, j64, loss = pl.pallas_call(
        _dist_body,
        out_shape=(
            jax.ShapeDtypeStruct((G, B), jnp.float32),
            jax.ShapeDtypeStruct((G, B), jnp.int32),
            jax.ShapeDtypeStruct((1, 1), jnp.float32),
        ),
        out_specs=(
            pl.BlockSpec(memory_space=pltpu.VMEM),
            pl.BlockSpec(memory_space=pltpu.VMEM),
            pl.BlockSpec(memory_space=pltpu.SMEM),
        ),
    )(input, weight)
    locs, idx = _sc_bmu(m64, j64, locations)
    return locs.reshape(B, 1, 2), loss.reshape(()), idx.reshape(B, 1)
